# baseline (speedup 1.0000x reference)
import functools

import jax
import jax.numpy as jnp
from jax import lax
from jax.experimental import pallas as pl
from jax.experimental.pallas import tpu as pltpu
from jax.experimental.pallas import tpu_sc as plsc

NB = 4
C = 128
D = 2
H = 200
W = 176
NV = 60000
NPOS = NB * D * H * W
NPOS2 = NPOS // 2

NT = 32
RNG = NPOS2 // NT
SENT = 1 << 30
CH = 6000
NCH = NV // CH
RB = 128
NBUF = 4
LROWS = RNG // RB + 2
PADR = NT * 128
HB = 8
NROW = NB * H * W


def _make_sc_body(half):
    def _sc_body(idx_hbm, vf_hbm, dense_hbm, valid_hbm,
                 idxbuf, inv_v, vbuf, plist, ilist, rowbuf,
                 gsem, ssem, csem):
        cid = lax.axis_index("c")
        sid = lax.axis_index("s")
        wid = sid * 2 + cid
        lbase = wid * RNG
        base = half * NPOS2 + lbase
        iota = lax.broadcasted_iota(jnp.int32, (16,), 0)

        sent_v = jnp.full((16,), SENT, jnp.int32)

        def init_b(j, _):
            for u in range(5):
                inv_v[pl.ds((j * 5 + u) * 16, 16)] = sent_v
            return 0

        lax.fori_loop(0, RNG // 80, init_b, 0)

        rng_u = jnp.uint32(RNG)

        def _issue_chunk(ci):
            pltpu.async_copy(idx_hbm.at[pl.ds(ci * CH, CH)],
                             idxbuf.at[pl.ds(lax.rem(ci, 2) * CH, CH)], csem)

        def chunk_b(ci, acc):
            @pl.when(ci + 1 < NCH)
            def _prefetch():
                _issue_chunk(ci + 1)

            pltpu.make_async_copy(idx_hbm.at[pl.ds(0, CH)],
                                  idxbuf.at[pl.ds(0, CH)], csem).wait()
            boff = lax.rem(ci, 2) * CH

            def win_b(j, acc):
                for u in range(5):
                    jj = j * 5 + u
                    v = idxbuf[pl.ds(boff + jj * 16, 16)]
                    li = v - base
                    m = plsc.bitcast(li, jnp.uint32) < rng_u
                    ids = ci * CH + jj * 16 + iota
                    plsc.store_scatter(inv_v, [li], ids, mask=m)
                    g = plsc.load_gather(inv_v, [li], mask=m)
                    acc = acc | (m & (g < ids))
                return acc

            return lax.fori_loop(0, CH // 80, win_b, acc)

        with jax.named_scope("p2_scan"):
            _issue_chunk(0)
            acc = lax.fori_loop(0, NCH, chunk_b, jnp.zeros((16,), jnp.bool_))
        dirty = jnp.max(plsc.all_reduce_population_count(acc)) > 0

        @pl.when(dirty)
        def _fixup():
            def fchunk_b(ci, _):
                pltpu.sync_copy(idx_hbm.at[pl.ds(ci * CH, CH)],
                                idxbuf.at[pl.ds(0, CH)])

                def win2(j, _):
                    v = idxbuf[pl.ds(j * 16, 16)]
                    li = v - base
                    m = plsc.bitcast(li, jnp.uint32) < rng_u
                    ids = ci * CH + j * 16 + iota
                    g = plsc.load_gather(inv_v, [li], mask=m)
                    m2 = m & (g < ids)

                    def cond(mm):
                        return jnp.max(
                            plsc.all_reduce_population_count(mm)) > 0

                    def fix(mm):
                        plsc.store_scatter(inv_v, [li], ids, mask=mm)
                        g2 = plsc.load_gather(inv_v, [li], mask=m)
                        return m & (g2 < ids)

                    lax.while_loop(cond, fix, m2)
                    return 0

                lax.fori_loop(0, CH // 16, win2, 0)
                return 0

            lax.fori_loop(0, NCH, fchunk_b, 0)

        def comp_b(j, cnt):
            for u in range(5):
                jj = j * 5 + u
                g = inv_v[pl.ds(jj * 16, 16)]
                m = g < NV
                vbuf[pl.ds(jj * 16, 16)] = jnp.where(m, 1, 0)
                c = plsc.cumsum(m.astype(jnp.int32))
                addr = cnt + c - 1
                row = lax.shift_right_logical(addr, 7)
                col = addr & 127
                pos = lbase + jj * 16 + iota
                plsc.store_scatter(plist, [row, col], pos, mask=m)
                plsc.store_scatter(ilist, [row, col], g, mask=m)
                cnt = cnt + plsc.all_reduce_population_count(m)
            return cnt

        with jax.named_scope("p3_compress"):
            cnt = lax.fori_loop(0, RNG // 80, comp_b,
                                jnp.zeros((16,), jnp.int32))
        cnt_s = jnp.max(cnt)

        def pad_b(j, _):
            addr = cnt_s + j * 16 + iota
            row = lax.shift_right_logical(addr, 7)
            col = addr & 127
            pos = NPOS2 + wid * 128 + j * 16 + iota
            plsc.store_scatter(plist, [row, col], pos)
            plsc.store_scatter(ilist, [row, col],
                               jnp.zeros((16,), jnp.int32))
            return 0

        lax.fori_loop(0, RB // 16, pad_b, 0)
        trips = (cnt_s + RB - 1) // RB

        def _issue_gather(t):
            pltpu.async_copy(vf_hbm.at[ilist.at[t]],
                             rowbuf.at[lax.rem(t, NBUF)], gsem)

        def _drain(sem):
            pltpu.make_async_copy(vf_hbm.at[ilist.at[0]], rowbuf.at[0],
                                  sem).wait()

        def prol_b(t, _):
            _issue_gather(t)
            return 0

        lax.fori_loop(0, jnp.minimum(trips, 2), prol_b, 0)

        def trip_b(t, _):
            @pl.when(t >= 2)
            def _drain_old_scatter():
                _drain(ssem)

            @pl.when(t + 2 < trips)
            def _prefetch():
                _issue_gather(t + 2)

            _drain(gsem)
            pltpu.async_copy(rowbuf.at[lax.rem(t, NBUF)],
                             dense_hbm.at[plist.at[t]], ssem)
            return 0

        with jax.named_scope("p4_streams"):
            lax.fori_loop(0, trips, trip_b, 0)

        def epi_b(k, _):
            _drain(ssem)
            return 0

        lax.fori_loop(0, jnp.minimum(trips, 2), epi_b, 0)

        pltpu.sync_copy(vbuf, valid_hbm.at[pl.ds(lbase, RNG)])

    return _sc_body


def _make_sc(half):
    return functools.partial(
        pl.kernel,
        out_type=(
            jax.ShapeDtypeStruct((NPOS2 + PADR, C), jnp.float32),
            jax.ShapeDtypeStruct((NPOS2,), jnp.int32),
        ),
        mesh=plsc.VectorSubcoreMesh(core_axis_name="c", subcore_axis_name="s"),
        compiler_params=pltpu.CompilerParams(needs_layout_passes=False,
                                             use_tc_tiling_on_sc=True),
        scratch_types=(
            pltpu.VMEM((2 * CH,), jnp.int32),
            pltpu.VMEM((RNG,), jnp.int32),
            pltpu.VMEM((RNG,), jnp.int32),
            pltpu.VMEM((LROWS, RB), jnp.int32),
            pltpu.VMEM((LROWS, RB), jnp.int32),
            pltpu.VMEM((NBUF, RB, C), jnp.float32),
            pltpu.SemaphoreType.DMA,
            pltpu.SemaphoreType.DMA,
            pltpu.SemaphoreType.DMA,
        ),
    )(_make_sc_body(half))


_sc_half = (_make_sc(0), _make_sc(1))


def _il_body(x0_ref, x1_ref, v0_ref, v1_ref, o_ref):
    x0 = x0_ref[...]
    x1 = x1_ref[...]
    vm0 = (v0_ref[...].reshape(HB * W) != 0).astype(jnp.float32)[:, None]
    vm1 = (v1_ref[...].reshape(HB * W) != 0).astype(jnp.float32)[:, None]
    row = lax.broadcasted_iota(jnp.int32, (C, 2 * C), 0)
    col = lax.broadcasted_iota(jnp.int32, (C, 2 * C), 1)
    e0 = (col == 2 * row).astype(jnp.bfloat16)
    e1 = (col == 2 * row + 1).astype(jnp.bfloat16)

    def sel(x, e):
        hi = x.astype(jnp.bfloat16)
        lo = (x - hi.astype(jnp.float32)).astype(jnp.bfloat16)
        return (jnp.dot(hi, e, preferred_element_type=jnp.float32)
                + jnp.dot(lo, e, preferred_element_type=jnp.float32))

    o_ref[...] = sel(x0 * vm0, e0) + sel(x1 * vm1, e1)


def _il_body_alias(x0_ref, x1_ref, v0_ref, v1_ref, prev_ref, o_ref):
    del prev_ref
    _il_body(x0_ref, x1_ref, v0_ref, v1_ref, o_ref)


def _interleave(dense, valid3d, half, prev=None):
    nhb = H // HB
    in_specs = [
        pl.BlockSpec((HB * W, C), lambda n, hb: (2 * n * nhb + hb, 0)),
        pl.BlockSpec((HB * W, C), lambda n, hb: ((2 * n + 1) * nhb + hb, 0)),
        pl.BlockSpec((1, HB * W // 128, 128),
                     lambda n, hb: (2 * n * nhb + hb, 0, 0)),
        pl.BlockSpec((1, HB * W // 128, 128),
                     lambda n, hb: ((2 * n + 1) * nhb + hb, 0, 0)),
    ]
    out_spec = pl.BlockSpec(
        (HB * W, 2 * C), lambda n, hb: ((half * 2 + n) * nhb + hb, 0))
    args = (dense, dense, valid3d, valid3d)
    kwargs = {}
    body = _il_body
    if prev is not None:
        in_specs = in_specs + [pl.BlockSpec(memory_space=pltpu.HBM)]
        args = args + (prev,)
        kwargs["input_output_aliases"] = {4: 0}
        body = _il_body_alias
    return pl.pallas_call(
        body,
        grid=(2, nhb),
        in_specs=in_specs,
        out_specs=out_spec,
        out_shape=jax.ShapeDtypeStruct((NROW, 2 * C), jnp.float32),
        **kwargs,
    )(*args)


@jax.jit
def kernel(voxel_features, voxel_indices):
    d0, v0 = _sc_half[0](voxel_indices, voxel_features)
    d1, v1 = _sc_half[1](voxel_indices, voxel_features)
    nv3 = (NPOS2 // 1408, HB * W // 128, 128)
    p0 = _interleave(d0, v0.reshape(nv3), 0)
    p1 = _interleave(d1, v1.reshape(nv3), 1, prev=p0)
    return jnp.transpose(p1.reshape(NB, H, W, D * C), (0, 3, 1, 2))


# speedup vs baseline: 1.3241x; 1.3241x over previous
import functools

import jax
import jax.numpy as jnp
from jax import lax
from jax.experimental import pallas as pl
from jax.experimental.pallas import tpu as pltpu
from jax.experimental.pallas import tpu_sc as plsc

NB = 4
C = 128
D = 2
H = 200
W = 176
NV = 60000
NPOS = NB * D * H * W

NT = 32
RNG = NPOS // NT
SENT = 1 << 30
CH = 6000
NCH = NV // CH
RB = 128
NBUF = 4
LROWS = RNG // RB + 2
PADR = NT * 128
HB = 8


def _sc_body(idx_hbm, vf_hbm, dense_hbm, valid_hbm,
             idxbuf, inv_v, vbuf, plist, ilist, rowbuf,
             gsem, ssem, csem):
    cid = lax.axis_index("c")
    sid = lax.axis_index("s")
    wid = sid * 2 + cid
    base = wid * RNG
    iota = lax.broadcasted_iota(jnp.int32, (16,), 0)

    sent_v = jnp.full((16,), SENT, jnp.int32)

    def init_b(j, _):
        for u in range(5):
            inv_v[pl.ds((j * 5 + u) * 16, 16)] = sent_v
        return 0

    with jax.named_scope("p1_init"):
        lax.fori_loop(0, RNG // 80, init_b, 0)

    rng_u = jnp.uint32(RNG)

    def _issue_chunk(ci):
        pltpu.async_copy(idx_hbm.at[pl.ds(ci * CH, CH)],
                         idxbuf.at[pl.ds(lax.rem(ci, 2) * CH, CH)], csem)

    def chunk_b(ci, acc):
        @pl.when(ci + 1 < NCH)
        def _prefetch():
            _issue_chunk(ci + 1)

        pltpu.make_async_copy(idx_hbm.at[pl.ds(0, CH)],
                              idxbuf.at[pl.ds(0, CH)], csem).wait()
        boff = lax.rem(ci, 2) * CH

        def win_b(j, acc):
            for u in range(5):
                jj = j * 5 + u
                v = idxbuf[pl.ds(boff + jj * 16, 16)]
                li = v - base
                m = plsc.bitcast(li, jnp.uint32) < rng_u
                ids = ci * CH + jj * 16 + iota
                plsc.store_scatter(inv_v, [li], ids, mask=m)
                g = plsc.load_gather(inv_v, [li], mask=m)
                acc = acc | (m & (g < ids))
            return acc

        return lax.fori_loop(0, CH // 80, win_b, acc)

    with jax.named_scope("p2_scan"):
        _issue_chunk(0)
        acc = lax.fori_loop(0, NCH, chunk_b, jnp.zeros((16,), jnp.bool_))
    dirty = jnp.max(plsc.all_reduce_population_count(acc)) > 0

    @pl.when(dirty)
    def _fixup():
        def fchunk_b(ci, _):
            pltpu.sync_copy(idx_hbm.at[pl.ds(ci * CH, CH)],
                            idxbuf.at[pl.ds(0, CH)])

            def win2(j, _):
                v = idxbuf[pl.ds(j * 16, 16)]
                li = v - base
                m = plsc.bitcast(li, jnp.uint32) < rng_u
                ids = ci * CH + j * 16 + iota
                g = plsc.load_gather(inv_v, [li], mask=m)
                m2 = m & (g < ids)

                def cond(mm):
                    return jnp.max(plsc.all_reduce_population_count(mm)) > 0

                def fix(mm):
                    plsc.store_scatter(inv_v, [li], ids, mask=mm)
                    g2 = plsc.load_gather(inv_v, [li], mask=m)
                    return m & (g2 < ids)

                lax.while_loop(cond, fix, m2)
                return 0

            lax.fori_loop(0, CH // 16, win2, 0)
            return 0

        lax.fori_loop(0, NCH, fchunk_b, 0)

    def comp_b(j, cnt):
        for u in range(5):
            jj = j * 5 + u
            g = inv_v[pl.ds(jj * 16, 16)]
            m = g < NV
            vbuf[pl.ds(jj * 16, 16)] = jnp.where(m, 1, 0)
            c = plsc.cumsum(m.astype(jnp.int32))
            addr = cnt + c - 1
            row = lax.shift_right_logical(addr, 7)
            col = addr & 127
            pos = base + jj * 16 + iota
            plsc.store_scatter(plist, [row, col], pos, mask=m)
            plsc.store_scatter(ilist, [row, col], g, mask=m)
            cnt = cnt + plsc.all_reduce_population_count(m)
        return cnt

    with jax.named_scope("p3_compress"):
        cnt = lax.fori_loop(0, RNG // 80, comp_b, jnp.zeros((16,), jnp.int32))
    cnt_s = jnp.max(cnt)

    def pad_b(j, _):
        addr = cnt_s + j * 16 + iota
        row = lax.shift_right_logical(addr, 7)
        col = addr & 127
        pos = NPOS + wid * 128 + j * 16 + iota
        plsc.store_scatter(plist, [row, col], pos)
        plsc.store_scatter(ilist, [row, col], jnp.zeros((16,), jnp.int32))
        return 0

    lax.fori_loop(0, RB // 16, pad_b, 0)
    trips = (cnt_s + RB - 1) // RB

    def _issue_gather(t):
        pltpu.async_copy(vf_hbm.at[ilist.at[t]],
                         rowbuf.at[lax.rem(t, NBUF)], gsem)

    def _drain(sem):
        pltpu.make_async_copy(vf_hbm.at[ilist.at[0]], rowbuf.at[0],
                              sem).wait()

    def prol_b(t, _):
        _issue_gather(t)
        return 0

    lax.fori_loop(0, jnp.minimum(trips, 2), prol_b, 0)

    def trip_b(t, _):
        @pl.when(t >= 2)
        def _drain_old_scatter():
            _drain(ssem)

        @pl.when(t + 2 < trips)
        def _prefetch():
            _issue_gather(t + 2)

        _drain(gsem)
        pltpu.async_copy(rowbuf.at[lax.rem(t, NBUF)],
                         dense_hbm.at[plist.at[t]], ssem)
        return 0

    with jax.named_scope("p4_streams"):
        lax.fori_loop(0, trips, trip_b, 0)

    def epi_b(k, _):
        _drain(ssem)
        return 0

    lax.fori_loop(0, jnp.minimum(trips, 2), epi_b, 0)

    with jax.named_scope("p5_valid"):
        pltpu.sync_copy(vbuf, valid_hbm.at[pl.ds(base, RNG)])


_sc_scatter = functools.partial(
    pl.kernel,
    out_type=(
        jax.ShapeDtypeStruct((NPOS + PADR, C), jnp.float32),
        jax.ShapeDtypeStruct((NPOS,), jnp.int32),
    ),
    mesh=plsc.VectorSubcoreMesh(core_axis_name="c", subcore_axis_name="s"),
    compiler_params=pltpu.CompilerParams(needs_layout_passes=False,
                                         use_tc_tiling_on_sc=True),
    scratch_types=(
        pltpu.VMEM((2 * CH,), jnp.int32),
        pltpu.VMEM((RNG,), jnp.int32),
        pltpu.VMEM((RNG,), jnp.int32),
        pltpu.VMEM((LROWS, RB), jnp.int32),
        pltpu.VMEM((LROWS, RB), jnp.int32),
        pltpu.VMEM((NBUF, RB, C), jnp.float32),
        pltpu.SemaphoreType.DMA,
        pltpu.SemaphoreType.DMA,
        pltpu.SemaphoreType.DMA,
    ),
)(_sc_body)


def _il_body(x0_ref, x1_ref, v0_ref, v1_ref, o_ref):
    x0 = x0_ref[...]
    x1 = x1_ref[...]
    vm0 = (v0_ref[...].reshape(HB * W) != 0).astype(jnp.float32)[:, None]
    vm1 = (v1_ref[...].reshape(HB * W) != 0).astype(jnp.float32)[:, None]
    row = lax.broadcasted_iota(jnp.int32, (C, 2 * C), 0)
    col = lax.broadcasted_iota(jnp.int32, (C, 2 * C), 1)
    e0 = (col == 2 * row).astype(jnp.bfloat16)
    e1 = (col == 2 * row + 1).astype(jnp.bfloat16)

    def sel(x, e):
        hi = x.astype(jnp.bfloat16)
        lo = (x - hi.astype(jnp.float32)).astype(jnp.bfloat16)
        return (jnp.dot(hi, e, preferred_element_type=jnp.float32)
                + jnp.dot(lo, e, preferred_element_type=jnp.float32))

    o_ref[...] = sel(x0 * vm0, e0) + sel(x1 * vm1, e1)


def _interleave(dense, valid3d):
    nhb = H // HB
    return pl.pallas_call(
        _il_body,
        grid=(NB, nhb),
        in_specs=[
            pl.BlockSpec((HB * W, C), lambda n, hb: (2 * n * nhb + hb, 0)),
            pl.BlockSpec((HB * W, C),
                         lambda n, hb: ((2 * n + 1) * nhb + hb, 0)),
            pl.BlockSpec((1, HB * W // 128, 128),
                         lambda n, hb: (2 * n * nhb + hb, 0, 0)),
            pl.BlockSpec((1, HB * W // 128, 128),
                         lambda n, hb: ((2 * n + 1) * nhb + hb, 0, 0)),
        ],
        out_specs=pl.BlockSpec((HB * W, 2 * C), lambda n, hb: (n * nhb + hb, 0)),
        out_shape=jax.ShapeDtypeStruct((NB * H * W, 2 * C), jnp.float32),
    )(dense, dense, valid3d, valid3d)


@jax.jit
def kernel(voxel_features, voxel_indices):
    dense, valid = _sc_scatter(voxel_indices, voxel_features)
    phys = _interleave(
        dense, valid.reshape(D * NB * H // HB, HB * W // 128, 128))
    return jnp.transpose(phys.reshape(NB, H, W, D * C), (0, 3, 1, 2))
